# Initial kernel scaffold; baseline (speedup 1.0000x reference)
import jax
import jax.numpy as jnp
from jax.experimental import pallas as pl


def kernel(table, token_ids_a, token_ids_b, alpha):
    raise NotImplementedError("write your pallas kernel here")


# speedup vs baseline: 3.6019x; 3.6019x over previous
import jax
import jax.numpy as jnp
from jax import lax
from jax.experimental import pallas as pl
from jax.experimental.pallas import tpu as pltpu
from jax.experimental.pallas import tpu_sc as plsc

VOCAB = 100000
DIM = 128
NB = 16384
KTOP = 8
LANES = 16
NCORE = 2
NSUB = 16
NW = NCORE * NSUB
PPW = NB // NW
CHA = 128
NREG = DIM // LANES
NSLICE = 8
SLICE_R = VOCAB // NSLICE
PPT = NB // NSUB
MAXM = 2 * PPT
MAXMP = MAXM + LANES
CHB = 16
UZERO = 2 * NB


def _fill_zeros(buf):
  z = jnp.zeros((LANES,), jnp.float32)
  for i in range(CHB):
    for j in range(NREG):
      buf[i, pl.ds(j * LANES, LANES)] = z


def _rsqrt16(x):
  xi = plsc.bitcast(x, jnp.int32)
  yi = jnp.int32(0x5F3759DF) - lax.shift_right_logical(xi, 1)
  y = plsc.bitcast(yi, jnp.float32)
  for _ in range(3):
    y = y * (1.5 - 0.5 * x * y * y)
  return y


def _sort16(v):
  return lax.sort(v, dimension=0, num_keys=1)


def _merge16(a, b):
  return _sort16(jnp.maximum(a, lax.rev(b, (0,))))


def _top16(vals):
  t = [_sort16(v) for v in vals]
  while len(t) > 1:
    t = [_merge16(t[2 * i], t[2 * i + 1]) for i in range(len(t) // 2)]
  return t[0]


def _phase_a_body(table, ida, idb, alphav, u_out,
                  ida_v, idb_v, al_v, ea, eb, ua, ub, zb):
  cid = lax.axis_index("c")
  sid = lax.axis_index("s")
  wid = sid * NCORE + cid
  base = wid * PPW
  pltpu.sync_copy(ida.at[pl.ds(base, PPW)], ida_v)
  pltpu.sync_copy(idb.at[pl.ds(base, PPW)], idb_v)
  pltpu.sync_copy(alphav, al_v)
  alpha16 = al_v[...]

  _fill_zeros(zb)

  @pl.when(wid == 0)
  def _():
    pltpu.sync_copy(zb, u_out.at[pl.ds(UZERO, CHB)])

  iot = lax.iota(jnp.int32, LANES)
  topmask = iot >= (LANES - KTOP)
  inf16 = jnp.full((LANES,), jnp.inf, jnp.float32)

  for ci in range(PPW // CHA):
    ia = ida_v.at[pl.ds(ci * CHA, CHA)]
    ib = idb_v.at[pl.ds(ci * CHA, CHA)]
    pltpu.sync_copy(table.at[ia], ea)
    pltpu.sync_copy(table.at[ib], eb)

    def pair_body(p, carry):
      a = [ea[p, pl.ds(j * LANES, LANES)] for j in range(NREG)]
      b = [eb[p, pl.ds(j * LANES, LANES)] for j in range(NREG)]
      al = [jnp.abs(a[j] * b[j]) for j in range(NREG)]
      d = [a[j] - b[j] for j in range(NREG)]
      ss = d[0] * d[0]
      for j in range(1, NREG):
        ss = ss + d[j] * d[j]
      tot = jnp.sum(ss)
      x16 = jnp.full((LANES,), tot, jnp.float32)
      y16 = _rsqrt16(x16)
      t16 = _top16(al)
      thr = jnp.min(jnp.where(topmask, t16, inf16))
      thr16 = jnp.full((LANES,), thr, jnp.float32)
      s16 = alpha16 * y16
      for j in range(NREG):
        u = jnp.where(al[j] >= thr16, s16 * d[j], 0.0)
        ua[p, pl.ds(j * LANES, LANES)] = u
        ub[p, pl.ds(j * LANES, LANES)] = -u
      return carry

    lax.fori_loop(0, CHA, pair_body, 0)
    pltpu.sync_copy(ua, u_out.at[pl.ds(base + ci * CHA, CHA)])
    pltpu.sync_copy(ub, u_out.at[pl.ds(NB + base + ci * CHA, CHA)])


def _phase_b_body(table, ida, idb, u_in, out,
                  ida_v, idb_v, rows_l, items_l, zb, ubuf, tbuf, dbuf, obuf,
                  sh_delta):
  cid = lax.axis_index("c")
  sid = lax.axis_index("s")
  tbase = sid * PPT
  pltpu.sync_copy(ida.at[pl.ds(tbase, PPT)], ida_v)
  pltpu.sync_copy(idb.at[pl.ds(tbase, PPT)], idb_v)
  _fill_zeros(zb)
  iot = lax.iota(jnp.int32, LANES)

  for sl in range(NSLICE // NCORE):
    s = sl * NCORE + cid
    lo = s * SLICE_R
    lo16 = jnp.full((LANES,), lo, jnp.int32)
    uz16 = jnp.full((LANES,), UZERO, jnp.int32)

    def fill_body(q, carry):
      rows_l[pl.ds(q * LANES, LANES)] = lo16
      items_l[pl.ds(q * LANES, LANES)] = uz16
      return carry
    lax.fori_loop(0, MAXMP // LANES, fill_body, 0)

    def make_scan(idv, item_off):
      def ch_body(ch, cnt):
        r = idv[pl.ds(ch * LANES, LANES)]
        m = (r >= lo16) & (r < lo16 + SLICE_R)
        pos = cnt + plsc.cumsum(m.astype(jnp.int32)) - 1
        plsc.store_scatter(rows_l, [pos], r, mask=m)
        it = item_off + ch * LANES + iot
        plsc.store_scatter(items_l, [pos], it, mask=m)
        return cnt + plsc.all_reduce_population_count(m)
      return ch_body

    cnt = jnp.zeros((LANES,), jnp.int32)
    cnt = lax.fori_loop(0, PPT // LANES, make_scan(ida_v, tbase), cnt)
    cnt = lax.fori_loop(0, PPT // LANES, make_scan(idb_v, NB + tbase), cnt)
    nch = (jnp.max(cnt) + (LANES - 1)) // LANES

    @pl.when(sid == 0)
    def _():
      pltpu.sync_copy(zb, sh_delta.at[pl.ds(0, CHB)])

    def zch(i, carry):
      lrow = rows_l[pl.ds(i * LANES, LANES)] - lo16
      pltpu.sync_copy(zb, sh_delta.at[lrow])
      return carry
    lax.fori_loop(0, nch, zch, 0)
    plsc.subcore_barrier()

    def ach(i, carry):
      itv = items_l[pl.ds(i * LANES, LANES)]
      lrow = rows_l[pl.ds(i * LANES, LANES)] - lo16
      pltpu.sync_copy(u_in.at[itv], ubuf)
      pltpu.sync_copy(ubuf, sh_delta.at[lrow], add=True)
      return carry
    lax.fori_loop(0, nch, ach, 0)
    plsc.subcore_barrier()

    def pch(i, carry):
      grow = rows_l[pl.ds(i * LANES, LANES)]
      lrow = grow - lo16
      pltpu.sync_copy(table.at[grow], tbuf)
      pltpu.sync_copy(sh_delta.at[lrow], dbuf)
      for rr in range(CHB):
        for j in range(NREG):
          obuf[rr, pl.ds(j * LANES, LANES)] = (
              tbuf[rr, pl.ds(j * LANES, LANES)]
              + dbuf[rr, pl.ds(j * LANES, LANES)])
      pltpu.sync_copy(obuf, out.at[grow])
      return carry
    lax.fori_loop(0, nch, pch, 0)
    plsc.subcore_barrier()


def _make_kernels():
  mesh = plsc.VectorSubcoreMesh(core_axis_name="c", subcore_axis_name="s",
                                num_cores=NCORE, num_subcores=NSUB)
  params = pltpu.CompilerParams(needs_layout_passes=False)
  phase_a = pl.kernel(
      _phase_a_body,
      out_type=jax.ShapeDtypeStruct((2 * NB + CHB, DIM), jnp.float32),
      mesh=mesh,
      compiler_params=params,
      scratch_types=[
          pltpu.VMEM((PPW,), jnp.int32),
          pltpu.VMEM((PPW,), jnp.int32),
          pltpu.VMEM((LANES,), jnp.float32),
          pltpu.VMEM((CHA, DIM), jnp.float32),
          pltpu.VMEM((CHA, DIM), jnp.float32),
          pltpu.VMEM((CHA, DIM), jnp.float32),
          pltpu.VMEM((CHA, DIM), jnp.float32),
          pltpu.VMEM((CHB, DIM), jnp.float32),
      ],
  )
  phase_b = pl.kernel(
      _phase_b_body,
      out_type=(),
      mesh=mesh,
      compiler_params=params,
      scratch_types=[
          pltpu.VMEM((PPT,), jnp.int32),
          pltpu.VMEM((PPT,), jnp.int32),
          pltpu.VMEM((MAXMP,), jnp.int32),
          pltpu.VMEM((MAXMP,), jnp.int32),
          pltpu.VMEM((CHB, DIM), jnp.float32),
          pltpu.VMEM((CHB, DIM), jnp.float32),
          pltpu.VMEM((CHB, DIM), jnp.float32),
          pltpu.VMEM((CHB, DIM), jnp.float32),
          pltpu.VMEM((CHB, DIM), jnp.float32),
          pltpu.VMEM_SHARED((SLICE_R, DIM), jnp.float32),
      ],
  )
  return phase_a, phase_b


_PHASE_A, _PHASE_B = None, None


def kernel(table, token_ids_a, token_ids_b, alpha):
  global _PHASE_A, _PHASE_B
  if _PHASE_A is None:
    _PHASE_A, _PHASE_B = _make_kernels()
  alpha16 = jnp.broadcast_to(alpha.astype(jnp.float32), (LANES,))
  u = _PHASE_A(table, token_ids_a, token_ids_b, alpha16)
  out_ref = jax.new_ref(table)
  _PHASE_B(table, token_ids_a, token_ids_b, u, out_ref)
  return jax.freeze(out_ref)
